# Initial kernel scaffold; baseline (speedup 1.0000x reference)
import jax
import jax.numpy as jnp
from jax.experimental import pallas as pl


def kernel(x, edge_index, edge_attr, Wm0, bm0, Wm1, bm1, Wm2, bm2, Wm3, bm3, Wr0, br0, Wr1, br1, Wr2, br2, Wr3, br3):
    raise NotImplementedError("write your pallas kernel here")


# speedup vs baseline: 2.3163x; 2.3163x over previous
import functools

import jax
import jax.numpy as jnp
from jax import lax
from jax.experimental import pallas as pl
from jax.experimental.pallas import tpu as pltpu
from jax.experimental.pallas import tpu_sc as plsc

F32 = jnp.float32

NC = 2
NS = 16
NW = NC * NS


def _lrelu(h):
    return jnp.where(h > 0, h, 0.2 * h)


def _prep_kernel(x_r, wa_r, wb_r, xs_r, xd_r):
    x = x_r[...]
    xs_r[...] = jnp.dot(x, wa_r[...], preferred_element_type=F32)
    xd_r[...] = jnp.dot(x, wb_r[...], preferred_element_type=F32)


def _tc_prep(x, wa, wb):
    n = x.shape[0]
    return pl.pallas_call(
        _prep_kernel,
        out_shape=[
            jax.ShapeDtypeStruct((n, 32), F32),
            jax.ShapeDtypeStruct((n, 32), F32),
        ],
    )(x, wa, wb)


def _sc_gather(xs, xd, src, dst):
    e = src.shape[0]
    epw = e // NW
    gch = 80
    nch = epw // gch
    mesh = plsc.VectorSubcoreMesh(core_axis_name="c", subcore_axis_name="s")

    @functools.partial(
        pl.kernel,
        out_type=jax.ShapeDtypeStruct((e, 32), F32),
        mesh=mesh,
        scratch_types=[
            pltpu.VMEM((gch,), jnp.int32),
            pltpu.VMEM((gch,), jnp.int32),
            pltpu.VMEM((gch, 32), F32),
            pltpu.VMEM((gch, 32), F32),
            pltpu.SemaphoreType.DMA,
            pltpu.SemaphoreType.DMA,
        ],
        compiler_params=pltpu.CompilerParams(use_tc_tiling_on_sc=False),
    )
    def k(xs_h, xd_h, src_h, dst_h, g_h, idx_s, idx_d, row_s, row_d, sem_a, sem_b):
        wid = lax.axis_index("s") * NC + lax.axis_index("c")
        base = wid * epw

        def body(c, carry):
            off = base + c * gch
            pltpu.sync_copy(src_h.at[pl.ds(off, gch)], idx_s)
            pltpu.sync_copy(dst_h.at[pl.ds(off, gch)], idx_d)
            ca = pltpu.async_copy(xs_h.at[idx_s], row_s, sem_a)
            cb = pltpu.async_copy(xd_h.at[idx_d], row_d, sem_b)
            ca.wait()
            cb.wait()
            for i in range(gch):
                for j in range(2):
                    sl = pl.ds(j * 16, 16)
                    row_s[i, sl] = row_s[i, sl] + row_d[i, sl]
            pltpu.sync_copy(row_s, g_h.at[pl.ds(off, gch)])
            return carry

        lax.fori_loop(0, nch, body, 0)

    return k(xs, xd, src, dst)


def _edge_kernel(g_r, ea_r, w0c_r, b0_r, w1_r, b1_r, w2_r, b2_r, w3_r, b3_r,
                 o1_r, o2_r):
    h1 = _lrelu(g_r[...] + jnp.dot(ea_r[...], w0c_r[...],
                                   preferred_element_type=F32) + b0_r[...])
    h2 = _lrelu(jnp.dot(h1, w1_r[...], preferred_element_type=F32) + b1_r[...])
    h3 = _lrelu(jnp.dot(h2, w2_r[...], preferred_element_type=F32) + b2_r[...])
    m = jnp.dot(h3, w3_r[...], preferred_element_type=F32) + b3_r[...]
    gate = jax.nn.sigmoid(m[:, 0:1])
    o1_r[...] = m[:, 16:32] * gate
    o2_r[...] = m[:, 32:48] * gate


def _tc_edge(g, ea, w0c, b0, w1, b1, w2, b2, w3p, b3p):
    e = g.shape[0]
    be = 8000
    grid = (e // be,)
    wspec = lambda a: pl.BlockSpec(a.shape, lambda i: (0,) * a.ndim)
    return pl.pallas_call(
        _edge_kernel,
        grid=grid,
        in_specs=[
            pl.BlockSpec((be, 32), lambda i: (i, 0)),
            pl.BlockSpec((be, 16), lambda i: (i, 0)),
            wspec(w0c), wspec(b0), wspec(w1), wspec(b1),
            wspec(w2), wspec(b2), wspec(w3p), wspec(b3p),
        ],
        out_specs=[
            pl.BlockSpec((be, 16), lambda i: (i, 0)),
            pl.BlockSpec((be, 16), lambda i: (i, 0)),
        ],
        out_shape=[
            jax.ShapeDtypeStruct((e, 16), F32),
            jax.ShapeDtypeStruct((e, 16), F32),
        ],
    )(g, ea, w0c, b0, w1, b1, w2, b2, w3p, b3p)


def _sc_reduce(dstv, ef1, ef2f, n):
    e = dstv.shape[0]
    epw = e // NW
    rch = 80
    nch = epw // rch
    t10 = n * 10
    nrow_tile = n // NS
    zrows = 125
    mesh = plsc.VectorSubcoreMesh(core_axis_name="c", subcore_axis_name="s")

    @functools.partial(
        pl.kernel,
        out_type=(
            jax.ShapeDtypeStruct((NC, n, 16), F32),
            jax.ShapeDtypeStruct((NW, t10), F32),
        ),
        mesh=mesh,
        scratch_types=[
            pltpu.VMEM((rch,), jnp.int32),
            pltpu.VMEM((rch, 16), F32),
            pltpu.VMEM((rch * 16,), F32),
            pltpu.VMEM((t10 + 16,), F32),
            pltpu.VMEM((zrows, 16), F32),
            pltpu.VMEM_SHARED((n, 16), F32),
        ],
        compiler_params=pltpu.CompilerParams(use_tc_tiling_on_sc=False),
    )
    def k(dst_h, ef1_h, ef2_h, sum_h, max_h, idx_d, buf1, buf2, mtab,
          zbuf, stab):
        cid = lax.axis_index("c")
        sid = lax.axis_index("s")
        wid = sid * NC + cid
        base = wid * epw
        lanes = lax.iota(jnp.int32, 16)
        mask = lanes < 10
        neginf = jnp.full((16,), -jnp.inf, F32)
        zero16 = jnp.zeros((16,), F32)

        def initm(i, carry):
            mtab[pl.ds(i * 16, 16)] = neginf
            return carry

        lax.fori_loop(0, (t10 + 16) // 16, initm, 0)

        def initz(i, carry):
            zbuf[i, pl.ds(0, 16)] = zero16
            return carry

        lax.fori_loop(0, zrows, initz, 0)

        def zcopy(q, carry):
            pltpu.sync_copy(
                zbuf, stab.at[pl.ds(sid * nrow_tile + q * zrows, zrows)])
            return carry

        lax.fori_loop(0, nrow_tile // zrows, zcopy, 0)

        plsc.subcore_barrier()

        def body(c, carry):
            off = base + c * rch
            pltpu.sync_copy(dst_h.at[pl.ds(off, rch)], idx_d)
            pltpu.sync_copy(ef1_h.at[pl.ds(off, rch)], buf1)
            pltpu.sync_copy(ef2_h.at[pl.ds(off * 16, rch * 16)], buf2)
            pltpu.sync_copy(buf1, stab.at[idx_d], add=True)

            def group(gi, c2):
                dvec = idx_d[pl.ds(gi * 16, 16)]
                for l in range(16):
                    d = dvec[l]
                    a = d * 10
                    msg = buf2[pl.ds(gi * 256 + l * 16, 16)]
                    old = mtab[pl.ds(a, 16)]
                    mtab[pl.ds(a, 16)] = jnp.where(
                        mask, jnp.maximum(old, msg), old)
                return c2

            lax.fori_loop(0, rch // 16, group, 0)
            return carry

        lax.fori_loop(0, nch, body, 0)

        plsc.subcore_barrier()

        pltpu.sync_copy(
            stab.at[pl.ds(sid * nrow_tile, nrow_tile)],
            sum_h.at[cid, pl.ds(sid * nrow_tile, nrow_tile)])
        pltpu.sync_copy(mtab.at[pl.ds(0, t10)], max_h.at[wid])

    return k(dstv, ef1, ef2f)


def _node_kernel(x_r, s_r, m_r, wa_r, wb_r, wc_r, b0_r, w1_r, b1_r, w2_r,
                 b2_r, w3_r, b3_r, o_r):
    nf1 = s_r[0] + s_r[1]
    nf2 = jnp.max(m_r[...], axis=0)
    nf2 = jnp.where(jnp.isneginf(nf2), 0.0, nf2)
    h = _lrelu(jnp.dot(x_r[...], wa_r[...], preferred_element_type=F32)
               + jnp.dot(nf1, wb_r[...], preferred_element_type=F32)
               + jnp.dot(nf2, wc_r[...], preferred_element_type=F32)
               + b0_r[...])
    h = _lrelu(jnp.dot(h, w1_r[...], preferred_element_type=F32) + b1_r[...])
    h = _lrelu(jnp.dot(h, w2_r[...], preferred_element_type=F32) + b2_r[...])
    o_r[...] = jnp.dot(h, w3_r[...], preferred_element_type=F32) + b3_r[...]


def _tc_node(x, sump, maxp, wa, wb, wc, b0, w1, b1, w2, b2, w3, b3):
    n = x.shape[0]
    bn = 200
    grid = (n // bn,)
    wspec = lambda a: pl.BlockSpec(a.shape, lambda i: (0,) * a.ndim)
    return pl.pallas_call(
        _node_kernel,
        grid=grid,
        in_specs=[
            pl.BlockSpec((bn, 128), lambda i: (i, 0)),
            pl.BlockSpec((NC, bn, 16), lambda i: (0, i, 0)),
            pl.BlockSpec((NW, bn, 10), lambda i: (0, i, 0)),
            wspec(wa), wspec(wb), wspec(wc), wspec(b0),
            wspec(w1), wspec(b1), wspec(w2), wspec(b2), wspec(w3), wspec(b3),
        ],
        out_specs=pl.BlockSpec((bn, 128), lambda i: (i, 0)),
        out_shape=jax.ShapeDtypeStruct((n, 128), F32),
    )(x, sump, maxp, wa, wb, wc, b0, w1, b1, w2, b2, w3, b3)


def kernel(x, edge_index, edge_attr, Wm0, bm0, Wm1, bm1, Wm2, bm2, Wm3, bm3,
           Wr0, br0, Wr1, br1, Wr2, br2, Wr3, br3):
    n = x.shape[0]
    e = edge_index.shape[1]
    src = edge_index[0]
    dst = edge_index[1]

    wm0a, wm0b, wm0c = Wm0[:128], Wm0[128:256], Wm0[256:272]
    w3p = jnp.zeros((32, 48), F32)
    w3p = (w3p.at[:, 0].set(Wm3[:, 0])
              .at[:, 16:26].set(Wm3[:, 1:11])
              .at[:, 32:42].set(Wm3[:, 11:21]))
    b3p = jnp.zeros((48,), F32)
    b3p = (b3p.at[0].set(bm3[0])
              .at[16:26].set(bm3[1:11])
              .at[32:42].set(bm3[11:21]))
    wr0a = Wr0[:128]
    wr0b = jnp.zeros((16, 32), F32).at[:10].set(Wr0[128:138])
    wr0c = Wr0[138:148]

    r2 = lambda b: b.reshape(1, -1)

    xs, xd = _tc_prep(x, wm0a, wm0b)
    g = _sc_gather(xs, xd, src, dst)
    ef1, ef2 = _tc_edge(g, edge_attr, wm0c, r2(bm0), Wm1, r2(bm1),
                        Wm2, r2(bm2), w3p, r2(b3p))
    sump, maxp = _sc_reduce(dst, ef1, ef2.reshape(-1), n)
    out = _tc_node(x, sump, maxp.reshape(NW, n, 10), wr0a, wr0b, wr0c,
                   r2(br0), Wr1, r2(br1), Wr2, r2(br2), Wr3, r2(br3))
    return out


# speedup vs baseline: 2.6366x; 1.1383x over previous
import functools

import jax
import jax.numpy as jnp
from jax import lax
from jax.experimental import pallas as pl
from jax.experimental.pallas import tpu as pltpu
from jax.experimental.pallas import tpu_sc as plsc

F32 = jnp.float32

NC = 2
NS = 16
NW = NC * NS


def _lrelu(h):
    return jnp.where(h > 0, h, 0.2 * h)


def _prep_kernel(x_r, wa_r, wb_r, xs_r, xd_r):
    x = x_r[...]
    xs_r[...] = jnp.dot(x, wa_r[...], preferred_element_type=F32)
    xd_r[...] = jnp.dot(x, wb_r[...], preferred_element_type=F32)


def _tc_prep(x, wa, wb):
    n = x.shape[0]
    return pl.pallas_call(
        _prep_kernel,
        out_shape=[
            jax.ShapeDtypeStruct((n, 32), F32),
            jax.ShapeDtypeStruct((n, 32), F32),
        ],
    )(x, wa, wb)


def _sc_gather(xs, xd, src2d, dst2d):
    nrows, rw = src2d.shape
    e = nrows * rw
    rpt = nrows // NW
    kr = 5
    sb = kr * rw
    nsc = rpt // kr
    mesh = plsc.VectorSubcoreMesh(core_axis_name="c", subcore_axis_name="s")

    @functools.partial(
        pl.kernel,
        out_type=(
            jax.ShapeDtypeStruct((e, 32), F32),
            jax.ShapeDtypeStruct((e, 32), F32),
        ),
        mesh=mesh,
        scratch_types=[
            pltpu.VMEM((rpt, rw), jnp.int32),
            pltpu.VMEM((rpt, rw), jnp.int32),
            pltpu.VMEM((2, sb, 32), F32),
            pltpu.VMEM((2, sb, 32), F32),
            pltpu.SemaphoreType.DMA,
            pltpu.SemaphoreType.DMA,
            pltpu.SemaphoreType.DMA,
        ],
        compiler_params=pltpu.CompilerParams(use_tc_tiling_on_sc=False),
    )
    def k(xs_h, xd_h, src_h, dst_h, gs_h, gd_h, isall, idall, row_s, row_d,
          semg, semw0, semw1):
        wid = lax.axis_index("s") * NC + lax.axis_index("c")
        rbase = wid * rpt
        ebase = wid * rpt * rw
        semw = (semw0, semw1)
        pltpu.sync_copy(src_h.at[pl.ds(rbase, rpt)], isall)
        pltpu.sync_copy(dst_h.at[pl.ds(rbase, rpt)], idall)
        writes = []
        for c in range(nsc):
            b = c % 2
            if c >= 2:
                writes[c - 2][0].wait()
                writes[c - 2][1].wait()
            gets = []
            for j in range(kr):
                r = c * kr + j
                gets.append(pltpu.async_copy(
                    xs_h.at[isall.at[r]],
                    row_s.at[b, pl.ds(j * rw, rw)], semg))
                gets.append(pltpu.async_copy(
                    xd_h.at[idall.at[r]],
                    row_d.at[b, pl.ds(j * rw, rw)], semg))
            for g in gets:
                g.wait()
            off = ebase + c * sb
            writes.append((
                pltpu.async_copy(row_s.at[b], gs_h.at[pl.ds(off, sb)],
                                 semw[b]),
                pltpu.async_copy(row_d.at[b], gd_h.at[pl.ds(off, sb)],
                                 semw[b]),
            ))
        for c in (nsc - 2, nsc - 1):
            writes[c][0].wait()
            writes[c][1].wait()

    return k(xs, xd, src2d, dst2d)


def _edge_kernel(gs_r, gd_r, ea_r, w0c_r, b0_r, w1_r, b1_r, w2_r, b2_r, w3_r,
                 b3_r, wk_r, bk_r, o1_r, o2_r):
    h1 = _lrelu(gs_r[...] + gd_r[...] + jnp.dot(ea_r[...], w0c_r[...],
                                                preferred_element_type=F32)
                + b0_r[...])
    h2 = _lrelu(jnp.dot(h1, w1_r[...], preferred_element_type=F32) + b1_r[...])
    h3 = _lrelu(jnp.dot(h2, w2_r[...], preferred_element_type=F32) + b2_r[...])
    m = jnp.dot(h3, w3_r[...], preferred_element_type=F32) + b3_r[...]
    gate = jax.nn.sigmoid(jnp.dot(h3, wk_r[...], preferred_element_type=F32)
                          + bk_r[...])
    o1_r[...] = m[:, 0:16] * gate[:, 0:16]
    o2_r[...] = m[:, 16:32] * gate[:, 16:32]


def _tc_edge(gs, gd, ea, w0c, b0, w1, b1, w2, b2, w3p, b3p, wkp, bkp):
    e = gs.shape[0]
    be = 8000
    grid = (e // be,)
    wspec = lambda a: pl.BlockSpec(a.shape, lambda i: (0,) * a.ndim)
    return pl.pallas_call(
        _edge_kernel,
        grid=grid,
        in_specs=[
            pl.BlockSpec((be, 32), lambda i: (i, 0)),
            pl.BlockSpec((be, 32), lambda i: (i, 0)),
            pl.BlockSpec((be, 16), lambda i: (i, 0)),
            wspec(w0c), wspec(b0), wspec(w1), wspec(b1),
            wspec(w2), wspec(b2), wspec(w3p), wspec(b3p),
            wspec(wkp), wspec(bkp),
        ],
        out_specs=[
            pl.BlockSpec((be, 16), lambda i: (i, 0)),
            pl.BlockSpec((be, 16), lambda i: (i, 0)),
        ],
        out_shape=[
            jax.ShapeDtypeStruct((e, 16), F32),
            jax.ShapeDtypeStruct((e, 16), F32),
        ],
    )(gs, gd, ea, w0c, b0, w1, b1, w2, b2, w3p, b3p, wkp, bkp)


def _sc_reduce(dst2d, ef1, ef2f, n):
    nrows, rw = dst2d.shape
    e = nrows * rw
    rpt = nrows // NW
    kr = 5
    sb = kr * rw
    nsc = rpt // kr
    t10 = n * 10
    nrow_tile = n // NS
    zrows = 125
    ngr = rw // 16 + 1
    mesh = plsc.VectorSubcoreMesh(core_axis_name="c", subcore_axis_name="s")

    @functools.partial(
        pl.kernel,
        out_type=(
            jax.ShapeDtypeStruct((NC, n, 16), F32),
            jax.ShapeDtypeStruct((NW, t10), F32),
        ),
        mesh=mesh,
        scratch_types=[
            pltpu.VMEM((kr, rw), jnp.int32),
            pltpu.VMEM((sb, 16), F32),
            pltpu.VMEM((sb * 16,), F32),
            pltpu.VMEM((t10 + 16,), F32),
            pltpu.VMEM((zrows, 16), F32),
            pltpu.VMEM_SHARED((n, 16), F32),
            pltpu.SemaphoreType.DMA,
        ],
        compiler_params=pltpu.CompilerParams(use_tc_tiling_on_sc=False),
    )
    def k(dst_h, ef1_h, ef2_h, sum_h, max_h, idx2, buf1, buf2, mtab,
          zbuf, stab, sems):
        cid = lax.axis_index("c")
        sid = lax.axis_index("s")
        wid = sid * NC + cid
        rbase = wid * rpt
        ebase = wid * rpt * rw
        lanes = lax.iota(jnp.int32, 16)
        mask = lanes < 10
        neginf = jnp.full((16,), -jnp.inf, F32)
        zero16 = jnp.zeros((16,), F32)

        def initm(i, carry):
            mtab[pl.ds(i * 16, 16)] = neginf
            return carry

        lax.fori_loop(0, (t10 + 16) // 16, initm, 0)

        def initz(i, carry):
            zbuf[i, pl.ds(0, 16)] = zero16
            return carry

        lax.fori_loop(0, zrows, initz, 0)

        def zcopy(q, carry):
            pltpu.sync_copy(
                zbuf, stab.at[pl.ds(sid * nrow_tile + q * zrows, zrows)])
            return carry

        lax.fori_loop(0, nrow_tile // zrows, zcopy, 0)

        plsc.subcore_barrier()

        def body(c, carry):
            eoff = ebase + c * sb
            pltpu.sync_copy(dst_h.at[pl.ds(rbase + c * kr, kr)], idx2)
            pltpu.sync_copy(ef1_h.at[pl.ds(eoff, sb)], buf1)
            pltpu.sync_copy(ef2_h.at[pl.ds(eoff * 16, sb * 16)], buf2)
            scs = []
            for j in range(kr):
                scs.append(pltpu.async_copy(
                    buf1.at[pl.ds(j * rw, rw)], stab.at[idx2.at[j]], sems,
                    add=True))
            for r in range(kr):

                def group(g, c2, r=r):
                    start = jnp.minimum(g * 16, rw - 16)
                    dvec = idx2[r, pl.ds(start, 16)]
                    for l in range(16):
                        a = dvec[l] * 10
                        msg = buf2[pl.ds((r * rw + start + l) * 16, 16)]
                        old = mtab[pl.ds(a, 16)]
                        mtab[pl.ds(a, 16)] = jnp.where(
                            mask, jnp.maximum(old, msg), old)
                    return c2

                lax.fori_loop(0, ngr, group, 0)
            for sc in scs:
                sc.wait()
            return carry

        lax.fori_loop(0, nsc, body, 0)

        plsc.subcore_barrier()

        pltpu.sync_copy(
            stab.at[pl.ds(sid * nrow_tile, nrow_tile)],
            sum_h.at[cid, pl.ds(sid * nrow_tile, nrow_tile)])
        pltpu.sync_copy(mtab.at[pl.ds(0, t10)], max_h.at[wid])

    return k(dst2d, ef1, ef2f)


def _node_kernel(x_r, s_r, m_r, wa_r, wb_r, wc_r, b0_r, w1_r, b1_r, w2_r,
                 b2_r, w3_r, b3_r, o_r):
    nf1 = s_r[0] + s_r[1]
    nf2 = jnp.max(m_r[...], axis=0)
    nf2 = jnp.where(jnp.isneginf(nf2), 0.0, nf2)
    h = _lrelu(jnp.dot(x_r[...], wa_r[...], preferred_element_type=F32)
               + jnp.dot(nf1, wb_r[...], preferred_element_type=F32)
               + jnp.dot(nf2, wc_r[...], preferred_element_type=F32)
               + b0_r[...])
    h = _lrelu(jnp.dot(h, w1_r[...], preferred_element_type=F32) + b1_r[...])
    h = _lrelu(jnp.dot(h, w2_r[...], preferred_element_type=F32) + b2_r[...])
    o_r[...] = jnp.dot(h, w3_r[...], preferred_element_type=F32) + b3_r[...]


def _tc_node(x, sump, maxp, wa, wb, wc, b0, w1, b1, w2, b2, w3, b3):
    n = x.shape[0]
    bn = 200
    grid = (n // bn,)
    wspec = lambda a: pl.BlockSpec(a.shape, lambda i: (0,) * a.ndim)
    return pl.pallas_call(
        _node_kernel,
        grid=grid,
        in_specs=[
            pl.BlockSpec((bn, 128), lambda i: (i, 0)),
            pl.BlockSpec((NC, bn, 16), lambda i: (0, i, 0)),
            pl.BlockSpec((NW, bn, 10), lambda i: (0, i, 0)),
            wspec(wa), wspec(wb), wspec(wc), wspec(b0),
            wspec(w1), wspec(b1), wspec(w2), wspec(b2), wspec(w3), wspec(b3),
        ],
        out_specs=pl.BlockSpec((bn, 128), lambda i: (i, 0)),
        out_shape=jax.ShapeDtypeStruct((n, 128), F32),
    )(x, sump, maxp, wa, wb, wc, b0, w1, b1, w2, b2, w3, b3)


def kernel(x, edge_index, edge_attr, Wm0, bm0, Wm1, bm1, Wm2, bm2, Wm3, bm3,
           Wr0, br0, Wr1, br1, Wr2, br2, Wr3, br3):
    n = x.shape[0]
    e = edge_index.shape[1]
    src = edge_index[0]
    dst = edge_index[1]

    wm0a, wm0b, wm0c = Wm0[:128], Wm0[128:256], Wm0[256:272]
    w3p = jnp.zeros((32, 32), F32)
    w3p = (w3p.at[:, 0:10].set(Wm3[:, 1:11])
              .at[:, 16:26].set(Wm3[:, 11:21]))
    b3p = jnp.zeros((32,), F32)
    b3p = (b3p.at[0:10].set(bm3[1:11])
              .at[16:26].set(bm3[11:21]))
    wkp = jnp.tile(Wm3[:, 0:1], (1, 32))
    bkp = jnp.full((32,), bm3[0], F32)
    wr0a = Wr0[:128]
    wr0b = jnp.zeros((16, 32), F32).at[:10].set(Wr0[128:138])
    wr0c = Wr0[138:148]

    r2 = lambda b: b.reshape(1, -1)

    rw = 100
    src2d = src.reshape(-1, rw)
    dst2d = dst.reshape(-1, rw)

    xs, xd = _tc_prep(x, wm0a, wm0b)
    gs, gd = _sc_gather(xs, xd, src2d, dst2d)
    ef1, ef2 = _tc_edge(gs, gd, edge_attr, wm0c, r2(bm0), Wm1, r2(bm1),
                        Wm2, r2(bm2), w3p, r2(b3p), wkp, r2(bkp))
    sump, maxp = _sc_reduce(dst2d, ef1, ef2.reshape(-1), n)
    out = _tc_node(x, sump, maxp.reshape(NW, n, 10), wr0a, wr0b, wr0c,
                   r2(br0), Wr1, r2(br1), Wr2, r2(br2), Wr3, r2(br3))
    return out


# speedup vs baseline: 4.4588x; 1.6911x over previous
import functools

import jax
import jax.numpy as jnp
from jax import lax
from jax.experimental import pallas as pl
from jax.experimental.pallas import tpu as pltpu
from jax.experimental.pallas import tpu_sc as plsc

F32 = jnp.float32

NC = 2
NS = 16
NW = NC * NS


def _lrelu(h):
    return jnp.where(h > 0, h, 0.2 * h)


def _bd4(w):
    z = jnp.zeros_like(w)
    return jnp.block([[w, z, z, z], [z, w, z, z], [z, z, w, z], [z, z, z, w]])


def _prep_kernel(x_r, wa_r, wb_r, xs_r, xd_r):
    x = x_r[...]
    xs_r[...] = jnp.dot(x, wa_r[...], preferred_element_type=F32)
    xd_r[...] = jnp.dot(x, wb_r[...], preferred_element_type=F32)


def _tc_prep(x, wa, wb):
    n = x.shape[0]
    return pl.pallas_call(
        _prep_kernel,
        out_shape=[
            jax.ShapeDtypeStruct((n, 32), F32),
            jax.ShapeDtypeStruct((n, 32), F32),
        ],
    )(x, wa, wb)


def _sc_gather(xs, xd, src, dst):
    e = src.shape[0]
    epw = e // NW
    sb = 400
    gb = 80
    nsc = epw // sb
    nb = sb // gb
    mesh = plsc.VectorSubcoreMesh(core_axis_name="c", subcore_axis_name="s")

    @functools.partial(
        pl.kernel,
        out_type=jax.ShapeDtypeStruct((e // 4, 128), F32),
        mesh=mesh,
        scratch_types=[
            pltpu.VMEM((epw,), jnp.int32),
            pltpu.VMEM((epw,), jnp.int32),
            pltpu.VMEM((2, sb, 32), F32),
            pltpu.VMEM((2, sb, 32), F32),
            pltpu.VMEM((sb // 4, 128), F32),
            pltpu.SemaphoreType.DMA,
            pltpu.SemaphoreType.DMA,
        ],
        compiler_params=pltpu.CompilerParams(use_tc_tiling_on_sc=False),
    )
    def k(xs_h, xd_h, src_h, dst_h, g4_h, isall, idall, row_s, row_d, acc,
          semg, semw):
        wid = lax.axis_index("s") * NC + lax.axis_index("c")
        ebase = wid * epw
        pltpu.sync_copy(src_h.at[pl.ds(ebase, epw)], isall)
        pltpu.sync_copy(dst_h.at[pl.ds(ebase, epw)], idall)

        def issue(c):
            b = c % 2
            gets = []
            for j in range(nb):
                o = c * sb + j * gb
                gets.append(pltpu.async_copy(
                    xs_h.at[isall.at[pl.ds(o, gb)]],
                    row_s.at[b, pl.ds(j * gb, gb)], semg))
                gets.append(pltpu.async_copy(
                    xd_h.at[idall.at[pl.ds(o, gb)]],
                    row_d.at[b, pl.ds(j * gb, gb)], semg))
            return gets

        inflight = issue(0)
        wr = None
        for c in range(nsc):
            b = c % 2
            for g in inflight:
                g.wait()
            if c + 1 < nsc:
                inflight = issue(c + 1)
            if wr is not None:
                wr.wait()

            def add(i, carry, b=b):
                for j in range(4):
                    for kk in range(2):
                        acc[i, pl.ds(j * 32 + kk * 16, 16)] = (
                            row_s[b, 4 * i + j, pl.ds(kk * 16, 16)]
                            + row_d[b, 4 * i + j, pl.ds(kk * 16, 16)])
                return carry

            lax.fori_loop(0, sb // 4, add, 0)
            wr = pltpu.async_copy(
                acc, g4_h.at[pl.ds((ebase + c * sb) // 4, sb // 4)], semw)
        wr.wait()

    return k(xs, xd, src, dst)


def _edge_kernel(g4_r, ea4_r, w0_r, b0_r, w1_r, b1_r, w2_r, b2_r, w3_r, b3_r,
                 wk_r, bk_r, pad_r, o_r):
    h1 = _lrelu(g4_r[...] + jnp.dot(ea4_r[...], w0_r[...],
                                    preferred_element_type=F32) + b0_r[...])
    h2 = _lrelu(jnp.dot(h1, w1_r[...], preferred_element_type=F32) + b1_r[...])
    h3 = _lrelu(jnp.dot(h2, w2_r[...], preferred_element_type=F32) + b2_r[...])
    m = jnp.dot(h3, w3_r[...], preferred_element_type=F32) + b3_r[...]
    gate = jax.nn.sigmoid(jnp.dot(h3, wk_r[...], preferred_element_type=F32)
                          + bk_r[...])
    o_r[...] = m * gate + pad_r[...]


def _tc_edge(g4, ea4, w0, b0, w1, b1, w2, b2, w3, b3, wk, bk, padv):
    e4 = g4.shape[0]
    be = 2000
    grid = (e4 // be,)
    wspec = lambda a: pl.BlockSpec(a.shape, lambda i: (0,) * a.ndim)
    return pl.pallas_call(
        _edge_kernel,
        grid=grid,
        in_specs=[
            pl.BlockSpec((be, 128), lambda i: (i, 0)),
            pl.BlockSpec((be, 64), lambda i: (i, 0)),
            wspec(w0), wspec(b0), wspec(w1), wspec(b1),
            wspec(w2), wspec(b2), wspec(w3), wspec(b3),
            wspec(wk), wspec(bk), wspec(padv),
        ],
        out_specs=pl.BlockSpec((be, 128), lambda i: (i, 0)),
        out_shape=jax.ShapeDtypeStruct((e4, 128), F32),
    )(g4, ea4, w0, b0, w1, b1, w2, b2, w3, b3, wk, bk, padv)


def _sc_reduce(dst2d, efm, n):
    nrows, rw = dst2d.shape
    e = nrows * rw
    rpt = nrows // NW
    kr = 2
    sb = kr * rw
    nsc = rpt // kr
    t10 = n * 10
    nrow_tile = n // NS
    zrows = 25
    ngr = rw // 16 + 1
    mesh = plsc.VectorSubcoreMesh(core_axis_name="c", subcore_axis_name="s")

    @functools.partial(
        pl.kernel,
        out_type=(
            jax.ShapeDtypeStruct((NC, n, 16), F32),
            jax.ShapeDtypeStruct((NW, t10), F32),
        ),
        mesh=mesh,
        scratch_types=[
            pltpu.VMEM((kr, rw), jnp.int32),
            pltpu.VMEM((sb, 32), F32),
            pltpu.VMEM((sb, 16), F32),
            pltpu.VMEM((t10 + 16,), F32),
            pltpu.VMEM((zrows, 16), F32),
            pltpu.VMEM_SHARED((n, 16), F32),
            pltpu.SemaphoreType.DMA,
        ],
        compiler_params=pltpu.CompilerParams(use_tc_tiling_on_sc=False),
    )
    def k(dst_h, efm_h, sum_h, max_h, idx2, buf, cbuf, mtab, zbuf, stab,
          sems):
        cid = lax.axis_index("c")
        sid = lax.axis_index("s")
        wid = sid * NC + cid
        rbase = wid * rpt
        ebase = wid * rpt * rw
        neginf = jnp.full((16,), -jnp.inf, F32)
        zero16 = jnp.zeros((16,), F32)

        def initm(i, carry):
            mtab[pl.ds(i * 16, 16)] = neginf
            return carry

        lax.fori_loop(0, (t10 + 16) // 16, initm, 0)

        def initz(i, carry):
            zbuf[i, pl.ds(0, 16)] = zero16
            return carry

        lax.fori_loop(0, zrows, initz, 0)

        def zcopy(q, carry):
            pltpu.sync_copy(
                zbuf, stab.at[pl.ds(sid * nrow_tile + q * zrows, zrows)])
            return carry

        lax.fori_loop(0, nrow_tile // zrows, zcopy, 0)

        plsc.subcore_barrier()

        def body(c, carry):
            eoff = ebase + c * sb
            pltpu.sync_copy(efm_h.at[pl.ds(eoff, sb)], buf)
            pltpu.sync_copy(dst_h.at[pl.ds(rbase + c * kr, kr)], idx2)

            def compact(e2, c2):
                cbuf[e2, pl.ds(0, 16)] = buf[e2, pl.ds(0, 16)]
                return c2

            lax.fori_loop(0, sb, compact, 0)
            scs = [
                pltpu.async_copy(cbuf.at[pl.ds(j * rw, rw)],
                                 stab.at[idx2.at[j]], sems, add=True)
                for j in range(kr)
            ]
            for r in range(kr):

                def group(g, c2, r=r):
                    start = jnp.minimum(g * 16, rw - 16)
                    dvec = idx2[r, pl.ds(start, 16)]
                    for l in range(16):
                        a = dvec[l] * 10
                        el = r * rw + start + l
                        mtab[pl.ds(a, 16)] = jnp.maximum(
                            mtab[pl.ds(a, 16)], buf[el, pl.ds(16, 16)])
                    return c2

                lax.fori_loop(0, ngr, group, 0)
            for s in scs:
                s.wait()
            return carry

        lax.fori_loop(0, nsc, body, 0)

        plsc.subcore_barrier()

        pltpu.sync_copy(
            stab.at[pl.ds(sid * nrow_tile, nrow_tile)],
            sum_h.at[cid, pl.ds(sid * nrow_tile, nrow_tile)])
        pltpu.sync_copy(mtab.at[pl.ds(0, t10)], max_h.at[wid])

    return k(dst2d, efm)


def _node_kernel(x_r, s_r, m_r, wa_r, wb_r, wc_r, b0_r, w1_r, b1_r,
                 w2_r, b2_r, w3_r, b3_r, o_r):
    nf1 = s_r[0] + s_r[1]
    nf2 = jnp.max(m_r[...], axis=0)
    nf2 = jnp.where(jnp.isneginf(nf2), 0.0, nf2)
    h = _lrelu(jnp.dot(x_r[...], wa_r[...], preferred_element_type=F32)
               + jnp.dot(nf1, wb_r[...], preferred_element_type=F32)
               + jnp.dot(nf2, wc_r[...], preferred_element_type=F32)
               + b0_r[...])
    h = _lrelu(jnp.dot(h, w1_r[...], preferred_element_type=F32) + b1_r[...])
    h = _lrelu(jnp.dot(h, w2_r[...], preferred_element_type=F32) + b2_r[...])
    o_r[...] = jnp.dot(h, w3_r[...], preferred_element_type=F32) + b3_r[...]


def _tc_node(x, sump, maxp, wa, wb, wc, b0, w1, b1, w2, b2, w3, b3):
    n = x.shape[0]
    bn = 200
    grid = (n // bn,)
    wspec = lambda a: pl.BlockSpec(a.shape, lambda i: (0,) * a.ndim)
    return pl.pallas_call(
        _node_kernel,
        grid=grid,
        in_specs=[
            pl.BlockSpec((bn, 128), lambda i: (i, 0)),
            pl.BlockSpec((NC, bn, 16), lambda i: (0, i, 0)),
            pl.BlockSpec((NW, bn, 10), lambda i: (0, i, 0)),
            wspec(wa), wspec(wb), wspec(wc), wspec(b0),
            wspec(w1), wspec(b1), wspec(w2), wspec(b2), wspec(w3), wspec(b3),
        ],
        out_specs=pl.BlockSpec((bn, 128), lambda i: (i, 0)),
        out_shape=jax.ShapeDtypeStruct((n, 128), F32),
    )(x, sump, maxp, wa, wb, wc, b0, w1, b1, w2, b2, w3, b3)


def kernel(x, edge_index, edge_attr, Wm0, bm0, Wm1, bm1, Wm2, bm2, Wm3, bm3,
           Wr0, br0, Wr1, br1, Wr2, br2, Wr3, br3):
    n = x.shape[0]
    e = edge_index.shape[1]
    src = edge_index[0]
    dst = edge_index[1]

    wm0a, wm0b, wm0c = Wm0[:128], Wm0[128:256], Wm0[256:272]
    w3p = jnp.zeros((32, 32), F32)
    w3p = (w3p.at[:, 0:10].set(Wm3[:, 1:11])
              .at[:, 16:26].set(Wm3[:, 11:21]))
    b3p = jnp.zeros((32,), F32)
    b3p = (b3p.at[0:10].set(bm3[1:11])
              .at[16:26].set(bm3[11:21]))
    wkp = jnp.tile(Wm3[:, 0:1], (1, 32))
    bkp = jnp.full((32,), bm3[0], F32)

    w0_4 = _bd4(wm0c)
    w1_4 = _bd4(Wm1)
    w2_4 = _bd4(Wm2)
    w3_4 = _bd4(w3p)
    wk_4 = _bd4(wkp)
    t4 = lambda b: jnp.tile(b, 4).reshape(1, -1)
    lane = jnp.arange(32)
    padv = jnp.tile(jnp.where(lane < 26, 0.0, -jnp.inf)
                    .at[10:16].set(0.0).astype(F32), 4).reshape(1, -1)

    wr0a = Wr0[:128]
    wr0b = jnp.zeros((16, 32), F32).at[:10].set(Wr0[128:138])
    wr0c = Wr0[138:148]

    r2 = lambda b: b.reshape(1, -1)

    xs, xd = _tc_prep(x, wm0a, wm0b)
    g4 = _sc_gather(xs, xd, src, dst)
    ea4 = edge_attr.reshape(e // 4, 64)
    efm = _tc_edge(g4, ea4, w0_4, t4(bm0), w1_4, t4(bm1), w2_4, t4(bm2),
                   w3_4, t4(b3p), wk_4, t4(bkp), padv)
    dst2d = dst.reshape(-1, 100)
    sump, maxp = _sc_reduce(dst2d, efm.reshape(e, 32), n)
    out = _tc_node(x, sump, maxp.reshape(NW, n, 10), wr0a, wr0b, wr0c,
                   r2(br0), Wr1, r2(br1), Wr2, r2(br2), Wr3, r2(br3))
    return out


# speedup vs baseline: 5.9657x; 1.3380x over previous
import functools

import jax
import jax.numpy as jnp
from jax import lax
from jax.experimental import pallas as pl
from jax.experimental.pallas import tpu as pltpu
from jax.experimental.pallas import tpu_sc as plsc

F32 = jnp.float32

NC = 2
NS = 16
NW = NC * NS


def _lrelu(h):
    return jnp.where(h > 0, h, 0.2 * h)


def _bd4(w):
    z = jnp.zeros_like(w)
    return jnp.block([[w, z, z, z], [z, w, z, z], [z, z, w, z], [z, z, z, w]])


def _prep_kernel(x_r, wa_r, wb_r, xs_r, xd_r):
    x = x_r[...]
    xs_r[...] = jnp.dot(x, wa_r[...], preferred_element_type=F32)
    xd_r[...] = jnp.dot(x, wb_r[...], preferred_element_type=F32)


def _tc_prep(x, wa, wb):
    n = x.shape[0]
    return pl.pallas_call(
        _prep_kernel,
        out_shape=[
            jax.ShapeDtypeStruct((n, 32), F32),
            jax.ShapeDtypeStruct((n, 32), F32),
        ],
    )(x, wa, wb)


def _sc_gather(xs, xd, src, dst):
    e = src.shape[0]
    epw = e // NW
    sb = 400
    gb = 80
    nsc = epw // sb
    nb = sb // gb
    mesh = plsc.VectorSubcoreMesh(core_axis_name="c", subcore_axis_name="s")

    @functools.partial(
        pl.kernel,
        out_type=jax.ShapeDtypeStruct((e // 4, 128), F32),
        mesh=mesh,
        scratch_types=[
            pltpu.VMEM((epw,), jnp.int32),
            pltpu.VMEM((epw,), jnp.int32),
            pltpu.VMEM((2, sb, 32), F32),
            pltpu.VMEM((2, sb, 32), F32),
            pltpu.VMEM((sb // 4, 128), F32),
            pltpu.SemaphoreType.DMA,
            pltpu.SemaphoreType.DMA,
        ],
        compiler_params=pltpu.CompilerParams(use_tc_tiling_on_sc=False),
    )
    def k(xs_h, xd_h, src_h, dst_h, g4_h, isall, idall, row_s, row_d, acc,
          semg, semw):
        wid = lax.axis_index("s") * NC + lax.axis_index("c")
        ebase = wid * epw
        pltpu.sync_copy(src_h.at[pl.ds(ebase, epw)], isall)
        pltpu.sync_copy(dst_h.at[pl.ds(ebase, epw)], idall)

        def issue(c):
            b = c % 2
            gets = []
            for j in range(nb):
                o = c * sb + j * gb
                gets.append(pltpu.async_copy(
                    xs_h.at[isall.at[pl.ds(o, gb)]],
                    row_s.at[b, pl.ds(j * gb, gb)], semg))
                gets.append(pltpu.async_copy(
                    xd_h.at[idall.at[pl.ds(o, gb)]],
                    row_d.at[b, pl.ds(j * gb, gb)], semg))
            return gets

        inflight = issue(0)
        wr = None
        for c in range(nsc):
            b = c % 2
            for g in inflight:
                g.wait()
            if c + 1 < nsc:
                inflight = issue(c + 1)
            if wr is not None:
                wr.wait()

            def add(i, carry, b=b):
                for j in range(4):
                    for kk in range(2):
                        acc[i, pl.ds(j * 32 + kk * 16, 16)] = (
                            row_s[b, 4 * i + j, pl.ds(kk * 16, 16)]
                            + row_d[b, 4 * i + j, pl.ds(kk * 16, 16)])
                return carry

            lax.fori_loop(0, sb // 4, add, 0)
            wr = pltpu.async_copy(
                acc, g4_h.at[pl.ds((ebase + c * sb) // 4, sb // 4)], semw)
        wr.wait()

    return k(xs, xd, src, dst)


def _edge_kernel(g4_r, ea4_r, w0_r, b0_r, w1_r, b1_r, w2_r, b2_r, w3_r, b3_r,
                 wk_r, bk_r, pad_r, o_r):
    h1 = _lrelu(g4_r[...] + jnp.dot(ea4_r[...], w0_r[...],
                                    preferred_element_type=F32) + b0_r[...])
    h2 = _lrelu(jnp.dot(h1, w1_r[...], preferred_element_type=F32) + b1_r[...])
    h3 = _lrelu(jnp.dot(h2, w2_r[...], preferred_element_type=F32) + b2_r[...])
    m = jnp.dot(h3, w3_r[...], preferred_element_type=F32) + b3_r[...]
    gate = jax.nn.sigmoid(jnp.dot(h3, wk_r[...], preferred_element_type=F32)
                          + bk_r[...])
    o_r[...] = m * gate + pad_r[...]


def _tc_edge(g4, ea4, w0, b0, w1, b1, w2, b2, w3, b3, wk, bk, padv):
    e4 = g4.shape[0]
    be = 2000
    grid = (e4 // be,)
    wspec = lambda a: pl.BlockSpec(a.shape, lambda i: (0,) * a.ndim)
    return pl.pallas_call(
        _edge_kernel,
        grid=grid,
        in_specs=[
            pl.BlockSpec((be, 128), lambda i: (i, 0)),
            pl.BlockSpec((be, 64), lambda i: (i, 0)),
            wspec(w0), wspec(b0), wspec(w1), wspec(b1),
            wspec(w2), wspec(b2), wspec(w3), wspec(b3),
            wspec(wk), wspec(bk), wspec(padv),
        ],
        out_specs=pl.BlockSpec((be, 128), lambda i: (i, 0)),
        out_shape=jax.ShapeDtypeStruct((e4, 128), F32),
    )(g4, ea4, w0, b0, w1, b1, w2, b2, w3, b3, wk, bk, padv)


def _sc_reduce(dst2d, efm, n):
    nrows, rw = dst2d.shape
    e = nrows * rw
    rpt = nrows // NW
    kr = 2
    sb = kr * rw
    nsc = rpt // kr
    t10 = n * 10
    t10p = ((t10 + 1023) // 1024) * 1024
    nrow_tile = n // NS
    zrows = 25
    ngr = rw // 16 + 1
    mesh = plsc.VectorSubcoreMesh(core_axis_name="c", subcore_axis_name="s")

    @functools.partial(
        pl.kernel,
        out_type=(
            jax.ShapeDtypeStruct((NC, n, 16), F32),
            jax.ShapeDtypeStruct((NW, t10p), F32),
        ),
        mesh=mesh,
        scratch_types=[
            pltpu.VMEM((kr, rw), jnp.int32),
            pltpu.VMEM((sb, 32), F32),
            pltpu.VMEM((sb, 16), F32),
            pltpu.VMEM((t10p + 16,), F32),
            pltpu.VMEM((zrows, 16), F32),
            pltpu.VMEM_SHARED((n, 16), F32),
            pltpu.SemaphoreType.DMA,
        ],
        compiler_params=pltpu.CompilerParams(use_tc_tiling_on_sc=False),
    )
    def k(dst_h, efm_h, sum_h, max_h, idx2, buf, cbuf, mtab, zbuf, stab,
          sems):
        cid = lax.axis_index("c")
        sid = lax.axis_index("s")
        wid = sid * NC + cid
        rbase = wid * rpt
        ebase = wid * rpt * rw
        neginf = jnp.full((16,), -jnp.inf, F32)
        zero16 = jnp.zeros((16,), F32)

        def initm(i, carry):
            mtab[pl.ds(i * 16, 16)] = neginf
            return carry

        lax.fori_loop(0, (t10p + 16) // 16, initm, 0)

        def initz(i, carry):
            zbuf[i, pl.ds(0, 16)] = zero16
            return carry

        lax.fori_loop(0, zrows, initz, 0)

        def zcopy(q, carry):
            pltpu.sync_copy(
                zbuf, stab.at[pl.ds(sid * nrow_tile + q * zrows, zrows)])
            return carry

        lax.fori_loop(0, nrow_tile // zrows, zcopy, 0)

        plsc.subcore_barrier()

        def body(c, carry):
            eoff = ebase + c * sb
            pltpu.sync_copy(efm_h.at[pl.ds(eoff, sb)], buf)
            pltpu.sync_copy(dst_h.at[pl.ds(rbase + c * kr, kr)], idx2)

            def compact(e2, c2):
                cbuf[e2, pl.ds(0, 16)] = buf[e2, pl.ds(0, 16)]
                return c2

            lax.fori_loop(0, sb, compact, 0)
            scs = [
                pltpu.async_copy(cbuf.at[pl.ds(j * rw, rw)],
                                 stab.at[idx2.at[j]], sems, add=True)
                for j in range(kr)
            ]
            for r in range(kr):

                def group(g, c2, r=r):
                    start = jnp.minimum(g * 16, rw - 16)
                    dvec = idx2[r, pl.ds(start, 16)]
                    for l in range(16):
                        a = dvec[l] * 10
                        el = r * rw + start + l
                        mtab[pl.ds(a, 16)] = jnp.maximum(
                            mtab[pl.ds(a, 16)], buf[el, pl.ds(16, 16)])
                    return c2

                lax.fori_loop(0, ngr, group, 0)
            for s in scs:
                s.wait()
            return carry

        lax.fori_loop(0, nsc, body, 0)

        plsc.subcore_barrier()

        pltpu.sync_copy(
            stab.at[pl.ds(sid * nrow_tile, nrow_tile)],
            sum_h.at[cid, pl.ds(sid * nrow_tile, nrow_tile)])
        pltpu.sync_copy(mtab.at[pl.ds(0, t10p)], max_h.at[wid])

    return k(dst2d, efm)


def _maxcomb_kernel(*refs):
    ins, o_r = refs[:-1], refs[-1]
    acc = ins[0][...]
    for r in ins[1:]:
        acc = jnp.maximum(acc, r[...])
    o_r[...] = acc


def _tc_maxcomb(mp2d):
    rows = mp2d.shape[0] // NW
    br = 56
    grid = (rows // br,)
    specs = [
        pl.BlockSpec((br, 128), functools.partial(
            lambda i, w: (w * (rows // br) + i, 0), w=w))
        for w in range(NW)
    ]
    return pl.pallas_call(
        _maxcomb_kernel,
        grid=grid,
        in_specs=specs,
        out_specs=pl.BlockSpec((br, 128), lambda i: (i, 0)),
        out_shape=jax.ShapeDtypeStruct((rows, 128), F32),
    )(*([mp2d] * NW))


def _node_kernel(x_r, s_r, m_r, wa_r, wb_r, wc_r, b0_r, w1_r, b1_r,
                 w2_r, b2_r, w3_r, b3_r, o_r):
    nf1 = s_r[0] + s_r[1]
    nf2 = m_r[...]
    nf2 = jnp.where(jnp.isneginf(nf2), 0.0, nf2)
    h = _lrelu(jnp.dot(x_r[...], wa_r[...], preferred_element_type=F32)
               + jnp.dot(nf1, wb_r[...], preferred_element_type=F32)
               + jnp.dot(nf2, wc_r[...], preferred_element_type=F32)
               + b0_r[...])
    h = _lrelu(jnp.dot(h, w1_r[...], preferred_element_type=F32) + b1_r[...])
    h = _lrelu(jnp.dot(h, w2_r[...], preferred_element_type=F32) + b2_r[...])
    o_r[...] = jnp.dot(h, w3_r[...], preferred_element_type=F32) + b3_r[...]


def _tc_node(x, sump, maxp, wa, wb, wc, b0, w1, b1, w2, b2, w3, b3):
    n = x.shape[0]
    bn = 200
    grid = (n // bn,)
    wspec = lambda a: pl.BlockSpec(a.shape, lambda i: (0,) * a.ndim)
    return pl.pallas_call(
        _node_kernel,
        grid=grid,
        in_specs=[
            pl.BlockSpec((bn, 128), lambda i: (i, 0)),
            pl.BlockSpec((NC, bn, 16), lambda i: (0, i, 0)),
            pl.BlockSpec((bn, 10), lambda i: (i, 0)),
            wspec(wa), wspec(wb), wspec(wc), wspec(b0),
            wspec(w1), wspec(b1), wspec(w2), wspec(b2), wspec(w3), wspec(b3),
        ],
        out_specs=pl.BlockSpec((bn, 128), lambda i: (i, 0)),
        out_shape=jax.ShapeDtypeStruct((n, 128), F32),
    )(x, sump, maxp, wa, wb, wc, b0, w1, b1, w2, b2, w3, b3)


def kernel(x, edge_index, edge_attr, Wm0, bm0, Wm1, bm1, Wm2, bm2, Wm3, bm3,
           Wr0, br0, Wr1, br1, Wr2, br2, Wr3, br3):
    n = x.shape[0]
    e = edge_index.shape[1]
    src = edge_index[0]
    dst = edge_index[1]

    wm0a, wm0b, wm0c = Wm0[:128], Wm0[128:256], Wm0[256:272]
    w3p = jnp.zeros((32, 32), F32)
    w3p = (w3p.at[:, 0:10].set(Wm3[:, 1:11])
              .at[:, 16:26].set(Wm3[:, 11:21]))
    b3p = jnp.zeros((32,), F32)
    b3p = (b3p.at[0:10].set(bm3[1:11])
              .at[16:26].set(bm3[11:21]))
    wkp = jnp.tile(Wm3[:, 0:1], (1, 32))
    bkp = jnp.full((32,), bm3[0], F32)

    w0_4 = _bd4(wm0c)
    w1_4 = _bd4(Wm1)
    w2_4 = _bd4(Wm2)
    w3_4 = _bd4(w3p)
    wk_4 = _bd4(wkp)
    t4 = lambda b: jnp.tile(b, 4).reshape(1, -1)
    lane = jnp.arange(32)
    padv = jnp.tile(jnp.where(lane < 26, 0.0, -jnp.inf)
                    .at[10:16].set(0.0).astype(F32), 4).reshape(1, -1)

    wr0a = Wr0[:128]
    wr0b = jnp.zeros((16, 32), F32).at[:10].set(Wr0[128:138])
    wr0c = Wr0[138:148]

    r2 = lambda b: b.reshape(1, -1)

    xs, xd = _tc_prep(x, wm0a, wm0b)
    g4 = _sc_gather(xs, xd, src, dst)
    ea4 = edge_attr.reshape(e // 4, 64)
    efm = _tc_edge(g4, ea4, w0_4, t4(bm0), w1_4, t4(bm1), w2_4, t4(bm2),
                   w3_4, t4(b3p), wk_4, t4(bkp), padv)
    dst2d = dst.reshape(-1, 100)
    sump, maxp = _sc_reduce(dst2d, efm.reshape(e, 32), n)
    mcomb = _tc_maxcomb(maxp.reshape(-1, 128))
    nf2 = mcomb.reshape(-1)[:n * 10].reshape(n, 10)
    out = _tc_node(x, sump, nf2, wr0a, wr0b, wr0c,
                   r2(br0), Wr1, r2(br1), Wr2, r2(br2), Wr3, r2(br3))
    return out
